# baseline (speedup 1.0000x reference)
import jax
import jax.numpy as jnp
from jax import lax
from jax.experimental import pallas as pl
from jax.experimental.pallas import tpu as pltpu
from jax.experimental.pallas import tpu_sc as plsc

NC = 2
NS = 16
F = 26
FH = F // 2
BPT = 1024
SBLK = 488
SREM = 5
G = 128
DB = 8
VB = 128


def _body(xidx, tview, dummy, out, idx_v, gout, spmem, gsem0, gsem1,
          osem0, osem1, ssem, ssem2):
    cid = lax.axis_index("c")
    sid = lax.axis_index("s")
    nvb = tview.shape[1]
    dpc = (tview.shape[0] * DB) // NC
    gsems = (gsem0, gsem1)
    osems = (osem0, osem1)
    njg = BPT // G
    d0 = cid * dpc

    @pl.loop(0, F)
    def _idx(f):
        pltpu.sync_copy(xidx.at[f * NS + sid], idx_v.at[f])

    def stage_fire(d):
        dg = d0 + d
        i = dg // DB
        r = dg % DB
        base = sid * SBLK

        @pl.loop(0, SBLK // 2, unroll=4)
        def _st(k):
            j = base + k
            pltpu.async_copy(tview.at[i, j, r],
                             spmem.at[pl.ds(j * VB, VB)], ssem)
            j2 = base + SBLK // 2 + k
            pltpu.async_copy(tview.at[i, j2, r],
                             spmem.at[pl.ds(j2 * VB, VB)], ssem2)

        @pl.when(sid == 0)
        def _():
            @pl.loop(0, SREM)
            def _st2(k):
                j = NS * SBLK + k
                pltpu.async_copy(tview.at[i, j, r],
                                 spmem.at[pl.ds(j * VB, VB)], ssem)

    def stage_wait(d):
        half = SBLK // 2 * VB
        pltpu.make_async_copy(
            dummy.at[pl.ds(0, half)],
            spmem.at[pl.ds(sid * SBLK * VB, half)], ssem).wait()
        pltpu.make_async_copy(
            dummy.at[pl.ds(0, half)],
            spmem.at[pl.ds(sid * SBLK * VB + half, half)], ssem2).wait()

        @pl.when(sid == 0)
        def _():
            pltpu.make_async_copy(
                dummy.at[pl.ds(0, SREM * VB)],
                spmem.at[pl.ds(NS * SBLK * VB, SREM * VB)], ssem).wait()

    def fire_gathers(s):
        @pl.loop(0, FH)
        def _(fh):
            for j in range(njg):
                pltpu.async_copy(
                    spmem.at[idx_v.at[s * FH + fh, pl.ds(j * G, G)]],
                    gout.at[s, fh, j], gsems[s])

    def drain_gathers(s):
        pltpu.make_async_copy(tview.at[0, pl.ds(0, FH)], gout.at[s],
                              gsems[s]).wait()

    def fire_out(s, d):
        dg = d0 + d
        i = dg // DB
        r = dg % DB

        @pl.loop(0, FH)
        def _(fh):
            pltpu.async_copy(
                gout.at[s, fh],
                out.at[s * FH + fh, i, pl.ds(sid * DB, DB), r], osems[s])

    def drain_out(s, d):
        pltpu.make_async_copy(tview.at[0, pl.ds(0, FH)], gout.at[s],
                              osems[s]).wait()

    stage_fire(0)
    stage_wait(0)
    plsc.subcore_barrier()

    @pl.loop(0, dpc)
    def _cols(d):
        @pl.when(d >= 1)
        def _():
            drain_out(0, d - 1)
            drain_out(1, d - 1)
        fire_gathers(0)
        fire_gathers(1)
        drain_gathers(0)
        fire_out(0, d)
        drain_gathers(1)
        fire_out(1, d)
        plsc.subcore_barrier()

        @pl.when(d + 1 < dpc)
        def _():
            stage_fire(d + 1)
            stage_wait(d + 1)
        plsc.subcore_barrier()

    drain_out(0, dpc - 1)
    drain_out(1, dpc - 1)


def kernel(x, table):
    B, FF = x.shape
    V, D = table.shape
    vpad = (-V) % VB
    nvb = (V + vpad) // VB
    ndb = D // DB
    xidx = jnp.swapaxes(x, 0, 1).astype(jnp.int32).reshape(FF * NS, B // NS)
    tpad = jnp.pad(table, ((0, vpad), (0, 0)))
    tview = (tpad.T.reshape(ndb, DB, nvb, VB).transpose(0, 2, 1, 3))
    dummy = jnp.zeros((SBLK * VB,), jnp.float32)
    mesh = plsc.VectorSubcoreMesh(core_axis_name="c", subcore_axis_name="s")
    out = pl.kernel(
        _body,
        out_type=jax.ShapeDtypeStruct((FF, ndb, B // VB, DB, VB),
                                      jnp.float32),
        mesh=mesh,
        scratch_types=[
            pltpu.VMEM((F, BPT), jnp.int32),
            pltpu.VMEM((2, FH, DB, G), jnp.float32),
            pltpu.VMEM_SHARED((nvb * VB,), jnp.float32),
            pltpu.SemaphoreType.DMA,
            pltpu.SemaphoreType.DMA,
            pltpu.SemaphoreType.DMA,
            pltpu.SemaphoreType.DMA,
            pltpu.SemaphoreType.DMA,
            pltpu.SemaphoreType.DMA,
        ],
        compiler_params=pltpu.CompilerParams(use_tc_tiling_on_sc=False),
    )(xidx, tview, dummy)
    return (out.transpose(0, 1, 3, 2, 4).reshape(FF, D, B)
            .transpose(2, 0, 1))
